# Initial kernel scaffold; baseline (speedup 1.0000x reference)
import jax
import jax.numpy as jnp
from jax.experimental import pallas as pl


def kernel(x_0, incidence_1, W1, W2, Wm1, bm1, Wm2, bm2, Wm3, bm3):
    raise NotImplementedError("write your pallas kernel here")


# speedup vs baseline: 13.2200x; 13.2200x over previous
import jax
import jax.numpy as jnp
from jax.experimental import pallas as pl


def _fused_kernel(x0_ref, b_ref, w1_ref, w2_ref, wm1_ref, bm1_ref,
                  wm2_ref, bm2_ref, wm3_ref, bm3_ref, out_ref):
    x0 = x0_ref[...]
    B = b_ref[...]

    contract0 = (((0,), (0,)), ((), ()))

    x1 = jax.lax.dot_general(B, x0, contract0,
                             preferred_element_type=jnp.float32)
    x0 = jnp.dot(B, jnp.dot(x1, w1_ref[...],
                            preferred_element_type=jnp.float32),
                 preferred_element_type=jnp.float32)

    x1 = jax.lax.dot_general(B, x0, contract0,
                             preferred_element_type=jnp.float32)
    x0 = jnp.dot(B, jnp.dot(x1, w2_ref[...],
                            preferred_element_type=jnp.float32),
                 preferred_element_type=jnp.float32)

    u = jnp.dot(wm2_ref[...], wm3_ref[...],
                preferred_element_type=jnp.float32)
    wm1 = wm1_ref[...]
    d = u.shape[0]
    av = jnp.dot(wm1[:d, :], u, preferred_element_type=jnp.float32)
    bv = jnp.dot(wm1[d:, :], u, preferred_element_type=jnp.float32)
    c = (jnp.dot(bm1_ref[...][None, :], u,
                 preferred_element_type=jnp.float32)[0, 0]
         + jnp.dot(bm2_ref[...][None, :], wm3_ref[...],
                   preferred_element_type=jnp.float32)[0, 0]
         + bm3_ref[0])

    a_col = jnp.dot(x0, av, preferred_element_type=jnp.float32)
    b_row = jax.lax.dot_general(bv, x1, (((0,), (1,)), ((), ())),
                                preferred_element_type=jnp.float32)

    vals = a_col + b_row + c
    out_ref[...] = jnp.where(B != 0, vals, 0.0)


def kernel(x_0, incidence_1, W1, W2, Wm1, bm1, Wm2, bm2, Wm3, bm3):
    n, m = incidence_1.shape
    return pl.pallas_call(
        _fused_kernel,
        out_shape=jax.ShapeDtypeStruct((n, m), jnp.float32),
    )(x_0, incidence_1, W1, W2, Wm1, bm1, Wm2, bm2, Wm3, bm3)
